# Initial kernel scaffold; baseline (speedup 1.0000x reference)
import functools

import jax
import jax.numpy as jnp
from jax import lax
from jax.experimental import pallas as pl
from jax.experimental.pallas import tpu as pltpu
from jax.experimental.pallas import tpu_sc as plsc

NN = 10000
NE = 320000
DIM = 128
NC, NS = 2, 16
NW = NC * NS
CHUNK = 128
NCH = -(-NE // (NW * CHUNK))
EPAD = NW * NCH * CHUNK
ACC_ROWS = 10240
TROWS = ACC_ROWS // NS
DUMMY = NN
DEGW = 8
RBLK = 1000

_mesh = lambda: plsc.VectorSubcoreMesh(core_axis_name="c", subcore_axis_name="s")


@functools.partial(
    pl.kernel,
    out_type=jax.ShapeDtypeStruct((NC, ACC_ROWS, DEGW), jnp.float32),
    mesh=_mesh(),
    scratch_types=[
        pltpu.VMEM((NCH, CHUNK), jnp.int32),
        pltpu.VMEM((CHUNK, DEGW), jnp.float32),
        pltpu.VMEM_SHARED((ACC_ROWS, DEGW), jnp.float32),
    ],
)
def _deg_sc(dst_hbm, ones_hbm, zeros_hbm, out_hbm, dst_v, ones_v, dacc):
    c = lax.axis_index("c")
    s = lax.axis_index("s")
    wid = c * NS + s
    pltpu.sync_copy(dst_hbm.at[wid], dst_v)
    pltpu.sync_copy(ones_hbm, ones_v)
    pltpu.sync_copy(zeros_hbm, dacc.at[pl.ds(s * TROWS, TROWS)])
    plsc.subcore_barrier()

    def step(j, carry):
        pltpu.sync_copy(ones_v, dacc.at[dst_v.at[j]], add=True)
        return carry

    lax.fori_loop(0, NCH, step, 0)
    plsc.subcore_barrier()
    pltpu.sync_copy(dacc.at[pl.ds(s * TROWS, TROWS)],
                    out_hbm.at[c, pl.ds(s * TROWS, TROWS)])


@functools.partial(
    pl.kernel,
    out_type=jax.ShapeDtypeStruct((NC, ACC_ROWS, DIM), jnp.float32),
    mesh=_mesh(),
    scratch_types=[
        pltpu.VMEM((NCH, CHUNK), jnp.int32),
        pltpu.VMEM((NCH, CHUNK), jnp.int32),
        pltpu.VMEM((CHUNK, DIM), jnp.float32),
        pltpu.VMEM_SHARED((ACC_ROWS, DIM), jnp.float32),
        pltpu.SemaphoreType.DMA,
    ],
)
def _agg_sc(p_hbm, src_hbm, dst_hbm, zeros_hbm, out_hbm,
            src_v, dst_v, rows_v, acc, sem):
    c = lax.axis_index("c")
    s = lax.axis_index("s")
    wid = c * NS + s
    pltpu.sync_copy(src_hbm.at[wid], src_v)
    pltpu.sync_copy(dst_hbm.at[wid], dst_v)
    pltpu.sync_copy(zeros_hbm, acc.at[pl.ds(s * TROWS, TROWS)])
    plsc.subcore_barrier()

    def step(j, carry):
        pltpu.async_copy(p_hbm.at[src_v.at[j]], rows_v, sem).wait()
        pltpu.sync_copy(rows_v, acc.at[dst_v.at[j]], add=True)
        return carry

    lax.fori_loop(0, NCH, step, 0)
    plsc.subcore_barrier()
    pltpu.sync_copy(acc.at[pl.ds(s * TROWS, TROWS)],
                    out_hbm.at[c, pl.ds(s * TROWS, TROWS)])


def _dis(d0_ref, d1_ref):
    deg = d0_ref[:, :1] + d1_ref[:, :1] + 1.0
    return lax.rsqrt(deg)


def _p1_tc(x_ref, w_ref, d0_ref, d1_ref, o_ref):
    h = jnp.dot(x_ref[:], w_ref[:], preferred_element_type=jnp.float32,
                precision=lax.Precision.HIGHEST)
    o_ref[:] = h * _dis(d0_ref, d1_ref)


def _mid_tc(s0_ref, s1_ref, p_ref, d0_ref, d1_ref, b_ref, w_ref, o_ref):
    dis = _dis(d0_ref, d1_ref)
    z = (s0_ref[:] + s1_ref[:] + p_ref[:]) * dis + b_ref[:]
    h = jnp.maximum(z, 0.0)
    o_ref[:] = jnp.dot(h, w_ref[:], preferred_element_type=jnp.float32,
                       precision=lax.Precision.HIGHEST) * dis


def _fin_tc(s0_ref, s1_ref, p_ref, d0_ref, d1_ref, b_ref, o_ref):
    z = (s0_ref[:] + s1_ref[:] + p_ref[:]) * _dis(d0_ref, d1_ref) + b_ref[:]
    m = jnp.max(z, axis=1, keepdims=True)
    lse = jnp.log(jnp.sum(jnp.exp(z - m), axis=1, keepdims=True)) + m
    o_ref[:] = z - lse


_row = pl.BlockSpec((RBLK, DIM), lambda i: (i, 0))
_deg_blk = pl.BlockSpec((RBLK, DEGW), lambda i: (i, 0))
_full = pl.BlockSpec((DIM, DIM), lambda i: (0, 0))
_bias = pl.BlockSpec((1, DIM), lambda i: (0, 0))
_GRID = NN // RBLK

_p1_call = pl.pallas_call(
    _p1_tc, grid=(_GRID,),
    in_specs=[_row, _full, _deg_blk, _deg_blk],
    out_specs=_row, out_shape=jax.ShapeDtypeStruct((NN, DIM), jnp.float32))

_mid_call = pl.pallas_call(
    _mid_tc, grid=(_GRID,),
    in_specs=[_row, _row, _row, _deg_blk, _deg_blk, _bias, _full],
    out_specs=_row, out_shape=jax.ShapeDtypeStruct((NN, DIM), jnp.float32))

_fin_call = pl.pallas_call(
    _fin_tc, grid=(_GRID,),
    in_specs=[_row, _row, _row, _deg_blk, _deg_blk, _bias],
    out_specs=_row, out_shape=jax.ShapeDtypeStruct((NN, DIM), jnp.float32))


def kernel(x, edge_index, W1, b1, W2, b2):
    src = edge_index[0].astype(jnp.int32)
    dst = edge_index[1].astype(jnp.int32)
    pad = EPAD - NE
    src_p = jnp.concatenate([src, jnp.zeros((pad,), jnp.int32)])
    dst_p = jnp.concatenate([dst, jnp.full((pad,), DUMMY, jnp.int32)])
    src_p = src_p.reshape(NW, NCH, CHUNK)
    dst_p = dst_p.reshape(NW, NCH, CHUNK)
    zerosF = jnp.zeros((TROWS, DIM), jnp.float32)
    zeros8 = jnp.zeros((TROWS, DEGW), jnp.float32)
    ones8 = jnp.ones((CHUNK, DEGW), jnp.float32)
    b1r = b1.reshape(1, DIM)
    b2r = b2.reshape(1, DIM)

    dpart = _deg_sc(dst_p, ones8, zeros8)
    d0, d1 = dpart[0], dpart[1]
    p1 = _p1_call(x, W1, d0, d1)
    s = _agg_sc(p1, src_p, dst_p, zerosF)
    p2 = _mid_call(s[0], s[1], p1, d0, d1, b1r, W2)
    s2 = _agg_sc(p2, src_p, dst_p, zerosF)
    return _fin_call(s2[0], s2[1], p2, d0, d1, b2r)


# speedup vs baseline: 12.1571x; 12.1571x over previous
import functools

import jax
import jax.numpy as jnp
from jax import lax
from jax.experimental import pallas as pl
from jax.experimental.pallas import tpu as pltpu
from jax.experimental.pallas import tpu_sc as plsc

NN = 10000
NE = 320000
DIM = 128
NC, NS = 2, 16
NW = NC * NS
CHUNK = 128
NCH = -(-NE // (NW * CHUNK))
EPAD = NW * NCH * CHUNK
ACC_ROWS = 10240
TROWS = ACC_ROWS // NS
DUMMY = NN
DEGW = 128
RBLK = 1000

_mesh = lambda: plsc.VectorSubcoreMesh(
    core_axis_name="c", subcore_axis_name="s", num_cores=NC, num_subcores=NS)


def _deg_body(dst_hbm, ones_hbm, zeros_hbm, out_hbm, dst_v, ones_v, dacc):
    c = lax.axis_index("c")
    s = lax.axis_index("s")
    wid = c * NS + s
    pltpu.sync_copy(dst_hbm.at[wid], dst_v)
    pltpu.sync_copy(ones_hbm, ones_v)
    pltpu.sync_copy(zeros_hbm, dacc.at[pl.ds(s * TROWS, TROWS)])
    plsc.subcore_barrier()

    def step(j, carry):
        pltpu.sync_copy(ones_v, dacc.at[dst_v.at[j]], add=True)
        return carry

    lax.fori_loop(0, NCH, step, 0)
    plsc.subcore_barrier()
    pltpu.sync_copy(dacc.at[pl.ds(s * TROWS, TROWS)],
                    out_hbm.at[c, pl.ds(s * TROWS, TROWS)])


def _make_deg(interpret=False):
    return pl.kernel(
        _deg_body,
        out_type=jax.ShapeDtypeStruct((NC, ACC_ROWS, DEGW), jnp.float32),
        mesh=_mesh(),
        scratch_types=[
            pltpu.VMEM((NCH, CHUNK), jnp.int32),
            pltpu.VMEM((CHUNK, DEGW), jnp.float32),
            pltpu.VMEM_SHARED((ACC_ROWS, DEGW), jnp.float32),
        ],
        interpret=interpret,
    )


_deg_sc = _make_deg()


def _agg_body(p_hbm, src_hbm, dst_hbm, zeros_hbm, out_hbm,
              src_v, dst_v, rows_v, acc, sem):
    c = lax.axis_index("c")
    s = lax.axis_index("s")
    wid = c * NS + s
    pltpu.sync_copy(src_hbm.at[wid], src_v)
    pltpu.sync_copy(dst_hbm.at[wid], dst_v)
    pltpu.sync_copy(zeros_hbm, acc.at[pl.ds(s * TROWS, TROWS)])
    plsc.subcore_barrier()

    def step(j, carry):
        pltpu.async_copy(p_hbm.at[src_v.at[j]], rows_v, sem).wait()
        pltpu.sync_copy(rows_v, acc.at[dst_v.at[j]], add=True)
        return carry

    lax.fori_loop(0, NCH, step, 0)
    plsc.subcore_barrier()
    pltpu.sync_copy(acc.at[pl.ds(s * TROWS, TROWS)],
                    out_hbm.at[c, pl.ds(s * TROWS, TROWS)])


def _make_agg(interpret=False):
    return pl.kernel(
        _agg_body,
        out_type=jax.ShapeDtypeStruct((NC, ACC_ROWS, DIM), jnp.float32),
        mesh=_mesh(),
        scratch_types=[
            pltpu.VMEM((NCH, CHUNK), jnp.int32),
            pltpu.VMEM((NCH, CHUNK), jnp.int32),
            pltpu.VMEM((CHUNK, DIM), jnp.float32),
            pltpu.VMEM_SHARED((ACC_ROWS, DIM), jnp.float32),
            pltpu.SemaphoreType.DMA,
        ],
        interpret=interpret,
    )


_agg_sc = _make_agg()


def _dis(d0_ref, d1_ref):
    deg = d0_ref[:, :1] + d1_ref[:, :1] + 1.0
    return lax.rsqrt(deg)


def _p1_tc(x_ref, w_ref, d0_ref, d1_ref, o_ref):
    h = jnp.dot(x_ref[:], w_ref[:], preferred_element_type=jnp.float32,
                precision=lax.Precision.HIGHEST)
    o_ref[:] = h * _dis(d0_ref, d1_ref)


def _mid_tc(s0_ref, s1_ref, p_ref, d0_ref, d1_ref, b_ref, w_ref, o_ref):
    dis = _dis(d0_ref, d1_ref)
    z = (s0_ref[:] + s1_ref[:] + p_ref[:]) * dis + b_ref[:]
    h = jnp.maximum(z, 0.0)
    o_ref[:] = jnp.dot(h, w_ref[:], preferred_element_type=jnp.float32,
                       precision=lax.Precision.HIGHEST) * dis


def _fin_tc(s0_ref, s1_ref, p_ref, d0_ref, d1_ref, b_ref, o_ref):
    z = (s0_ref[:] + s1_ref[:] + p_ref[:]) * _dis(d0_ref, d1_ref) + b_ref[:]
    m = jnp.max(z, axis=1, keepdims=True)
    lse = jnp.log(jnp.sum(jnp.exp(z - m), axis=1, keepdims=True)) + m
    o_ref[:] = z - lse


_row = pl.BlockSpec((RBLK, DIM), lambda i: (i, 0))
_deg_blk = pl.BlockSpec((RBLK, DEGW), lambda i: (i, 0))
_full = pl.BlockSpec((DIM, DIM), lambda i: (0, 0))
_bias = pl.BlockSpec((1, DIM), lambda i: (0, 0))
_GRID = NN // RBLK

_p1_call = pl.pallas_call(
    _p1_tc, grid=(_GRID,),
    in_specs=[_row, _full, _deg_blk, _deg_blk],
    out_specs=_row, out_shape=jax.ShapeDtypeStruct((NN, DIM), jnp.float32))

_mid_call = pl.pallas_call(
    _mid_tc, grid=(_GRID,),
    in_specs=[_row, _row, _row, _deg_blk, _deg_blk, _bias, _full],
    out_specs=_row, out_shape=jax.ShapeDtypeStruct((NN, DIM), jnp.float32))

_fin_call = pl.pallas_call(
    _fin_tc, grid=(_GRID,),
    in_specs=[_row, _row, _row, _deg_blk, _deg_blk, _bias],
    out_specs=_row, out_shape=jax.ShapeDtypeStruct((NN, DIM), jnp.float32))


def kernel(x, edge_index, W1, b1, W2, b2):
    src = edge_index[0].astype(jnp.int32)
    dst = edge_index[1].astype(jnp.int32)
    pad = EPAD - NE
    src_p = jnp.concatenate([src, jnp.zeros((pad,), jnp.int32)])
    dst_p = jnp.concatenate([dst, jnp.full((pad,), DUMMY, jnp.int32)])
    src_p = src_p.reshape(NW, NCH, CHUNK)
    dst_p = dst_p.reshape(NW, NCH, CHUNK)
    zerosF = jnp.zeros((TROWS, DIM), jnp.float32)
    zeros8 = jnp.zeros((TROWS, DEGW), jnp.float32)
    ones8 = jnp.ones((CHUNK, DEGW), jnp.float32)
    b1r = b1.reshape(1, DIM)
    b2r = b2.reshape(1, DIM)

    dpart = _deg_sc(dst_p, ones8, zeros8)
    d0, d1 = dpart[0], dpart[1]
    p1 = _p1_call(x, W1, d0, d1)
    s = _agg_sc(p1, src_p, dst_p, zerosF)
    p2 = _mid_call(s[0], s[1], p1, d0, d1, b1r, W2)
    s2 = _agg_sc(p2, src_p, dst_p, zerosF)
    return _fin_call(s2[0], s2[1], p2, d0, d1, b2r)
